# Initial kernel scaffold; baseline (speedup 1.0000x reference)
import jax
import jax.numpy as jnp
from jax.experimental import pallas as pl


def kernel(features, edge_index, batch_nodes, W1, b1, gamma1, W2, b2, gamma2, bn1_scale, bn1_bias, bn2_scale, bn2_bias):
    raise NotImplementedError("write your pallas kernel here")


# speedup vs baseline: 12.1529x; 12.1529x over previous
import functools

import jax
import jax.numpy as jnp
from jax import lax
from jax.experimental import pallas as pl
from jax.experimental.pallas import tpu as pltpu
from jax.experimental.pallas import tpu_sc as plsc

N = 10000
NP = 10240
E = 320000
D = 128
NC = 2
NS = 16
L = 16
NW = NC * NS
EP = 327680
EPW = EP // NW
CH = 80
NCHUNK = EPW // CH
RPS = NP // NS
IB = 16
NIB = NCHUNK // IB
BLK = 512
GRID = NP // BLK

@functools.cache
def _mesh():
    return plsc.VectorSubcoreMesh(core_axis_name="c", subcore_axis_name="s",
                                  num_cores=NC, num_subcores=NS)


def _deg_body(dst_hbm, degp_hbm, didx_v, ones_v, zb_v, deg_sp):
    c = lax.axis_index("c")
    s = lax.axis_index("s")
    w = c * NS + s

    @pl.loop(0, CH // L)
    def _(i):
        ones_v[pl.ds(i * L, L)] = jnp.ones((L,), jnp.float32)

    @pl.loop(0, RPS // L)
    def _(i):
        zb_v[pl.ds(i * L, L)] = jnp.zeros((L,), jnp.float32)

    pltpu.sync_copy(zb_v, deg_sp.at[pl.ds(s * RPS, RPS)])
    plsc.subcore_barrier()

    @pl.loop(0, NIB)
    def _(ob):
        pltpu.sync_copy(dst_hbm.at[w, ob], didx_v)

        @pl.loop(0, IB)
        def _(i):
            pltpu.sync_copy(ones_v, deg_sp.at[didx_v.at[i]], add=True)

    plsc.subcore_barrier()
    pltpu.sync_copy(deg_sp.at[pl.ds(s * RPS, RPS)],
                    degp_hbm.at[pl.ds(c * NP + s * RPS, RPS)])


def _sc_deg(dst_r):
    return pl.kernel(
        _deg_body,
        out_type=jax.ShapeDtypeStruct((NC * NP,), jnp.float32),
        mesh=_mesh(),
        compiler_params=pltpu.CompilerParams(needs_layout_passes=False),
        scratch_types=[
            pltpu.VMEM((IB, CH), jnp.int32),
            pltpu.VMEM((CH,), jnp.float32),
            pltpu.VMEM((RPS,), jnp.float32),
            pltpu.VMEM_SHARED((NP,), jnp.float32),
        ],
    )(dst_r)


def _agg1_body(y_hbm, src_hbm, dst_hbm, zeros_hbm, aggp_hbm,
               sidx_v, didx_v, rows0_v, rows1_v, agg_sp, sem0, sem1, semA):
    c = lax.axis_index("c")
    s = lax.axis_index("s")
    w = c * NS + s

    pltpu.sync_copy(zeros_hbm.at[pl.ds(s * RPS, RPS)],
                    agg_sp.at[pl.ds(s * RPS, RPS)])
    plsc.subcore_barrier()

    rows = (rows0_v, rows1_v)
    sems = (sem0, sem1)

    @pl.loop(0, NIB)
    def _(ob):
        pltpu.sync_copy(src_hbm.at[w, ob], sidx_v)
        pltpu.sync_copy(dst_hbm.at[w, ob], didx_v)

        cps = {0: pltpu.async_copy(y_hbm.at[sidx_v.at[0]], rows[0], sems[0])}
        scats = {}
        for i in range(IB):
            if i + 1 < IB:
                if i - 1 in scats:
                    scats.pop(i - 1).wait()
                cps[i + 1] = pltpu.async_copy(
                    y_hbm.at[sidx_v.at[i + 1]], rows[(i + 1) % 2],
                    sems[(i + 1) % 2])
            cps[i].wait()
            scats[i] = pltpu.async_copy(rows[i % 2],
                                        agg_sp.at[didx_v.at[i]], semA,
                                        add=True)
        for r in sorted(scats):
            scats.pop(r).wait()

    plsc.subcore_barrier()
    pltpu.sync_copy(agg_sp.at[pl.ds(s * RPS, RPS)],
                    aggp_hbm.at[pl.ds(c * NP + s * RPS, RPS)])


def _sc_agg1(y, src_r, dst_r, zeros):
    return pl.kernel(
        _agg1_body,
        out_type=jax.ShapeDtypeStruct((NC * NP, D), jnp.float32),
        mesh=_mesh(),
        compiler_params=pltpu.CompilerParams(needs_layout_passes=False),
        scratch_types=[
            pltpu.VMEM((IB, CH), jnp.int32),
            pltpu.VMEM((IB, CH), jnp.int32),
            pltpu.VMEM((CH, D), jnp.float32),
            pltpu.VMEM((CH, D), jnp.float32),
            pltpu.VMEM_SHARED((NP, D), jnp.float32),
            pltpu.SemaphoreType.DMA,
            pltpu.SemaphoreType.DMA,
            pltpu.SemaphoreType.DMA,
        ],
    )(y, src_r, dst_r, zeros)


def _agg2_weights(srow_v, drow_v, ng_v, dbuf_v, wbuf_v, lanes):
    @pl.loop(0, CH // L)
    def _(g):
        @pl.loop(0, L, unroll=4)
        def _(j):
            e = g * L + j
            d2 = jnp.zeros((L,), jnp.float32)
            for k in range(D // L):
                ak = srow_v[e, pl.ds(k * L, L)]
                bk = drow_v[e, pl.ds(k * L, L)]
                df = ak - bk
                d2 = d2 + df * df
            plsc.store_scatter(dbuf_v, [lanes * L + j], d2)

        tot = jnp.zeros((L,), jnp.float32)
        for l in range(L):
            tot = tot + dbuf_v[pl.ds(l * L, L)]
        wbuf_v[pl.ds(g * L, L)] = jnp.exp(ng_v[...] * tot)


def _agg2_scale(srow_v, wbuf_v):
    @pl.loop(0, CH, unroll=4)
    def _(e):
        we = plsc.load_gather(wbuf_v, [jnp.broadcast_to(e, (L,))])
        for k in range(D // L):
            srow_v[e, pl.ds(k * L, L)] = srow_v[e, pl.ds(k * L, L)] * we


def _agg2_body(x_hbm, src_hbm, dst_hbm, zeros_hbm, ng_hbm, aggp_hbm,
               sidx_v, didx_v, srow0_v, srow1_v, drow_v, ng_v, dbuf_v,
               wbuf_v, agg_sp, semS0, semS1, semD, semM0, semM1):
    c = lax.axis_index("c")
    s = lax.axis_index("s")
    w = c * NS + s

    pltpu.sync_copy(zeros_hbm.at[pl.ds(s * RPS, RPS)],
                    agg_sp.at[pl.ds(s * RPS, RPS)])
    plsc.subcore_barrier()

    pltpu.sync_copy(ng_hbm, ng_v)

    lanes = lax.iota(jnp.int32, L)

    @pl.loop(0, NIB)
    def _(ob):
        pltpu.sync_copy(src_hbm.at[w, ob], sidx_v)
        pltpu.sync_copy(dst_hbm.at[w, ob], didx_v)

        srows = (srow0_v, srow1_v)
        semSs = (semS0, semS1)
        semMs = (semM0, semM1)

        cpS = {0: pltpu.async_copy(x_hbm.at[sidx_v.at[0]], srows[0],
                                   semSs[0])}
        cpD = pltpu.async_copy(x_hbm.at[didx_v.at[0]], drow_v, semD)
        scat = None
        for i in range(IB):
            sr = srows[i % 2]
            cpS[i].wait()
            cpD.wait()
            _agg2_weights(sr, drow_v, ng_v, dbuf_v, wbuf_v, lanes)
            if i + 1 < IB:
                cpD = pltpu.async_copy(x_hbm.at[didx_v.at[i + 1]], drow_v,
                                       semD)
            if scat is not None:
                scat.wait()
            if i + 1 < IB:
                cpS[i + 1] = pltpu.async_copy(
                    x_hbm.at[sidx_v.at[i + 1]], srows[(i + 1) % 2],
                    semSs[(i + 1) % 2])
            _agg2_scale(sr, wbuf_v)
            scat = pltpu.async_copy(sr, agg_sp.at[didx_v.at[i]],
                                    semMs[i % 2], add=True)
        scat.wait()

    plsc.subcore_barrier()
    pltpu.sync_copy(agg_sp.at[pl.ds(s * RPS, RPS)],
                    aggp_hbm.at[pl.ds(c * NP + s * RPS, RPS)])


def _sc_agg2(x1, src_r, dst_r, zeros, ngamma):
    return pl.kernel(
        _agg2_body,
        out_type=jax.ShapeDtypeStruct((NC * NP, D), jnp.float32),
        mesh=_mesh(),
        compiler_params=pltpu.CompilerParams(needs_layout_passes=False),
        scratch_types=[
            pltpu.VMEM((IB, CH), jnp.int32),
            pltpu.VMEM((IB, CH), jnp.int32),
            pltpu.VMEM((CH, D), jnp.float32),
            pltpu.VMEM((CH, D), jnp.float32),
            pltpu.VMEM((CH, D), jnp.float32),
            pltpu.VMEM((L,), jnp.float32),
            pltpu.VMEM((L * L,), jnp.float32),
            pltpu.VMEM((CH,), jnp.float32),
            pltpu.VMEM_SHARED((NP, D), jnp.float32),
            pltpu.SemaphoreType.DMA,
            pltpu.SemaphoreType.DMA,
            pltpu.SemaphoreType.DMA,
            pltpu.SemaphoreType.DMA,
            pltpu.SemaphoreType.DMA,
        ],
    )(x1, src_r, dst_r, zeros, ngamma)


_BPW = 1024 // NW


def _sel_body(z_hbm, bidx_hbm, out_hbm, bidx_v, rows_v, sem):
    c = lax.axis_index("c")
    s = lax.axis_index("s")
    w = c * NS + s
    base = w * _BPW
    pltpu.sync_copy(bidx_hbm.at[pl.ds(base, _BPW)], bidx_v)
    pltpu.async_copy(z_hbm.at[bidx_v], rows_v, sem).wait()
    pltpu.sync_copy(rows_v, out_hbm.at[pl.ds(base, _BPW)])


def _sc_sel(z2, batch_nodes):
    return pl.kernel(
        _sel_body,
        out_type=jax.ShapeDtypeStruct((1024, D), jnp.float32),
        mesh=_mesh(),
        compiler_params=pltpu.CompilerParams(needs_layout_passes=False),
        scratch_types=[
            pltpu.VMEM((_BPW,), jnp.int32),
            pltpu.VMEM((_BPW, D), jnp.float32),
            pltpu.SemaphoreType.DMA,
        ],
    )(z2, batch_nodes)


def _prep1_body(deg_ref, x_ref, y_ref):
    d = jnp.maximum(deg_ref[0] + deg_ref[1], 1.0)
    s = lax.rsqrt(d)
    y_ref[...] = x_ref[...] * s


def _tc_prep1(degp, xp):
    return pl.pallas_call(
        _prep1_body,
        grid=(GRID,),
        in_specs=[
            pl.BlockSpec((NC, BLK, 1), lambda i: (0, i, 0)),
            pl.BlockSpec((BLK, D), lambda i: (i, 0)),
        ],
        out_specs=pl.BlockSpec((BLK, D), lambda i: (i, 0)),
        out_shape=jax.ShapeDtypeStruct((NP, D), jnp.float32),
    )(degp, xp)


def _mm_bn_body(deg_ref, agg_ref, x_ref, w_ref, b_ref, g_ref, z_ref, st_ref,
                *, self_scaled):
    i = pl.program_id(0)
    agg = agg_ref[0] + agg_ref[1]
    if self_scaled:
        d = jnp.maximum(deg_ref[0] + deg_ref[1], 1.0)
        s = lax.rsqrt(d)
        pre = g_ref[0, 0] * s * agg + x_ref[...]
    else:
        pre = agg + x_ref[...]
    z = jnp.dot(pre, w_ref[...], preferred_element_type=jnp.float32)
    z = z + b_ref[...]
    z_ref[...] = z

    rows = i * BLK + lax.broadcasted_iota(jnp.int32, (BLK, 1), 0)
    m = (rows < N).astype(jnp.float32)
    zm = z * m
    sums = jnp.sum(zm, axis=0, keepdims=True)
    sqs = jnp.sum(zm * zm, axis=0, keepdims=True)

    @pl.when(i == 0)
    def _():
        st_ref[...] = jnp.zeros_like(st_ref)

    st_ref[...] += jnp.concatenate([sums, sqs], axis=0)


def _tc_mm_bn(degp, aggp, x, W, b, g, self_scaled):
    body = functools.partial(_mm_bn_body, self_scaled=self_scaled)
    return pl.pallas_call(
        body,
        grid=(GRID,),
        in_specs=[
            pl.BlockSpec((NC, BLK, 1), lambda i: (0, i, 0)),
            pl.BlockSpec((NC, BLK, D), lambda i: (0, i, 0)),
            pl.BlockSpec((BLK, D), lambda i: (i, 0)),
            pl.BlockSpec((D, D), lambda i: (0, 0)),
            pl.BlockSpec((1, D), lambda i: (0, 0)),
            pl.BlockSpec((1, 1), lambda i: (0, 0)),
        ],
        out_specs=[
            pl.BlockSpec((BLK, D), lambda i: (i, 0)),
            pl.BlockSpec((2, D), lambda i: (0, 0)),
        ],
        out_shape=[
            jax.ShapeDtypeStruct((NP, D), jnp.float32),
            jax.ShapeDtypeStruct((2, D), jnp.float32),
        ],
    )(degp, aggp, x, W, b, g)


def _bn_relu_body(z_ref, st_ref, sc_ref, bi_ref, o_ref):
    mu = st_ref[0:1, :] / N
    var = st_ref[1:2, :] / N - mu * mu
    inv = lax.rsqrt(var + 1e-5)
    h = (z_ref[...] - mu) * inv * sc_ref[...] + bi_ref[...]
    o_ref[...] = jnp.maximum(h, 0.0)


def _tc_bn_relu(z, st, scale, bias):
    return pl.pallas_call(
        _bn_relu_body,
        grid=(GRID,),
        in_specs=[
            pl.BlockSpec((BLK, D), lambda i: (i, 0)),
            pl.BlockSpec((2, D), lambda i: (0, 0)),
            pl.BlockSpec((1, D), lambda i: (0, 0)),
            pl.BlockSpec((1, D), lambda i: (0, 0)),
        ],
        out_specs=pl.BlockSpec((BLK, D), lambda i: (i, 0)),
        out_shape=jax.ShapeDtypeStruct((NP, D), jnp.float32),
    )(z, st, scale, bias)


def _final_body(z_ref, st_ref, sc_ref, bi_ref, o_ref):
    mu = st_ref[0:1, :] / N
    var = st_ref[1:2, :] / N - mu * mu
    inv = lax.rsqrt(var + 1e-5)
    h = (z_ref[...] - mu) * inv * sc_ref[...] + bi_ref[...]
    h = jnp.maximum(h, 0.0)
    mx = jnp.max(h, axis=1, keepdims=True)
    ex = jnp.exp(h - mx)
    lse = jnp.log(jnp.sum(ex, axis=1, keepdims=True))
    o_ref[...] = h - mx - lse


def _tc_final(zsel, st, scale, bias):
    return pl.pallas_call(
        _final_body,
        grid=(2,),
        in_specs=[
            pl.BlockSpec((BLK, D), lambda i: (i, 0)),
            pl.BlockSpec((2, D), lambda i: (0, 0)),
            pl.BlockSpec((1, D), lambda i: (0, 0)),
            pl.BlockSpec((1, D), lambda i: (0, 0)),
        ],
        out_specs=pl.BlockSpec((BLK, D), lambda i: (i, 0)),
        out_shape=jax.ShapeDtypeStruct((1024, D), jnp.float32),
    )(zsel, st, scale, bias)


def kernel(features, edge_index, batch_nodes, W1, b1, gamma1, W2, b2, gamma2,
           bn1_scale, bn1_bias, bn2_scale, bn2_bias):
    f32 = jnp.float32
    xp = jnp.zeros((NP, D), f32).at[:N].set(features)
    k = jnp.arange(EP - E)
    pad_src = (N + k % (NP - N)).astype(jnp.int32)
    pad_dst = (N + (k + 120) % (NP - N)).astype(jnp.int32)
    src_r = jnp.concatenate(
        [edge_index[0].astype(jnp.int32), pad_src]).reshape(NW, NIB, IB, CH)
    dst_r = jnp.concatenate(
        [edge_index[1].astype(jnp.int32), pad_dst]).reshape(NW, NIB, IB, CH)
    zeros = jnp.zeros((NP, D), f32)
    g1 = jnp.reshape(gamma1.astype(f32), (1, 1))
    ngamma2 = jnp.broadcast_to(-gamma2.astype(f32), (L,))
    sc1 = jnp.reshape(bn1_scale, (1, D))
    bi1 = jnp.reshape(bn1_bias, (1, D))
    sc2 = jnp.reshape(bn2_scale, (1, D))
    bi2 = jnp.reshape(bn2_bias, (1, D))
    b1r = jnp.reshape(b1, (1, D))
    b2r = jnp.reshape(b2, (1, D))

    degp = _sc_deg(dst_r).reshape(NC, NP, 1)
    y = _tc_prep1(degp, xp)
    aggp = _sc_agg1(y, src_r, dst_r, zeros).reshape(NC, NP, D)
    z1, st1 = _tc_mm_bn(degp, aggp, xp, W1, b1r, g1, True)
    x1 = _tc_bn_relu(z1, st1, sc1, bi1)
    aggp2 = _sc_agg2(x1, src_r, dst_r, zeros, ngamma2).reshape(NC, NP, D)
    z2, st2 = _tc_mm_bn(degp, aggp2, x1, W2, b2r, g1, False)
    zsel = _sc_sel(z2, batch_nodes.astype(jnp.int32))
    return _tc_final(zsel, st2, sc2, bi2)


# speedup vs baseline: 12.6477x; 1.0407x over previous
import functools

import jax
import jax.numpy as jnp
from jax import lax
from jax.experimental import pallas as pl
from jax.experimental.pallas import tpu as pltpu
from jax.experimental.pallas import tpu_sc as plsc

N = 10000
NP = 10240
E = 320000
D = 128
NC = 2
NS = 16
L = 16
NW = NC * NS
EPW = E // NW
CH = 80
NCHUNK = EPW // CH
RPS = NP // NS
IB = 25
NIB = NCHUNK // IB
BLK = 512
GRID = NP // BLK

@functools.cache
def _mesh():
    return plsc.VectorSubcoreMesh(core_axis_name="c", subcore_axis_name="s",
                                  num_cores=NC, num_subcores=NS)


def _deg_body(dst_hbm, degp_hbm, didx_v, ones_v, zb_v, deg_sp):
    c = lax.axis_index("c")
    s = lax.axis_index("s")
    w = c * NS + s

    @pl.loop(0, CH // L)
    def _(i):
        ones_v[pl.ds(i * L, L)] = jnp.ones((L,), jnp.float32)

    @pl.loop(0, RPS // L)
    def _(i):
        zb_v[pl.ds(i * L, L)] = jnp.zeros((L,), jnp.float32)

    pltpu.sync_copy(zb_v, deg_sp.at[pl.ds(s * RPS, RPS)])
    plsc.subcore_barrier()

    @pl.loop(0, NIB)
    def _(ob):
        pltpu.sync_copy(dst_hbm.at[w, ob], didx_v)

        @pl.loop(0, IB)
        def _(i):
            pltpu.sync_copy(ones_v, deg_sp.at[didx_v.at[i]], add=True)

    plsc.subcore_barrier()
    pltpu.sync_copy(deg_sp.at[pl.ds(s * RPS, RPS)],
                    degp_hbm.at[pl.ds(c * NP + s * RPS, RPS)])


def _sc_deg(dst_r):
    return pl.kernel(
        _deg_body,
        out_type=jax.ShapeDtypeStruct((NC * NP,), jnp.float32),
        mesh=_mesh(),
        compiler_params=pltpu.CompilerParams(needs_layout_passes=False),
        scratch_types=[
            pltpu.VMEM((IB, CH), jnp.int32),
            pltpu.VMEM((CH,), jnp.float32),
            pltpu.VMEM((RPS,), jnp.float32),
            pltpu.VMEM_SHARED((NP,), jnp.float32),
        ],
    )(dst_r)


def _agg1_body(y_hbm, src_hbm, dst_hbm, zeros_hbm, aggp_hbm,
               sidx_v, didx_v, rows0_v, rows1_v, agg_sp, sem0, sem1, semA):
    c = lax.axis_index("c")
    s = lax.axis_index("s")
    w = c * NS + s

    pltpu.sync_copy(zeros_hbm.at[pl.ds(s * RPS, RPS)],
                    agg_sp.at[pl.ds(s * RPS, RPS)])
    plsc.subcore_barrier()

    rows = (rows0_v, rows1_v)
    sems = (sem0, sem1)

    @pl.loop(0, NIB)
    def _(ob):
        pltpu.sync_copy(src_hbm.at[w, ob], sidx_v)
        pltpu.sync_copy(dst_hbm.at[w, ob], didx_v)

        cps = {0: pltpu.async_copy(y_hbm.at[sidx_v.at[0]], rows[0], sems[0])}
        for i in range(IB):
            if i + 1 < IB:
                cps[i + 1] = pltpu.async_copy(
                    y_hbm.at[sidx_v.at[i + 1]], rows[(i + 1) % 2],
                    sems[(i + 1) % 2])
            cps[i].wait()
            pltpu.sync_copy(rows[i % 2], agg_sp.at[didx_v.at[i]], add=True)

    plsc.subcore_barrier()
    pltpu.sync_copy(agg_sp.at[pl.ds(s * RPS, RPS)],
                    aggp_hbm.at[pl.ds(c * NP + s * RPS, RPS)])


def _sc_agg1(y, src_r, dst_r, zeros):
    return pl.kernel(
        _agg1_body,
        out_type=jax.ShapeDtypeStruct((NC * NP, D), jnp.float32),
        mesh=_mesh(),
        compiler_params=pltpu.CompilerParams(needs_layout_passes=False),
        scratch_types=[
            pltpu.VMEM((IB, CH), jnp.int32),
            pltpu.VMEM((IB, CH), jnp.int32),
            pltpu.VMEM((CH, D), jnp.float32),
            pltpu.VMEM((CH, D), jnp.float32),
            pltpu.VMEM_SHARED((NP, D), jnp.float32),
            pltpu.SemaphoreType.DMA,
            pltpu.SemaphoreType.DMA,
            pltpu.SemaphoreType.DMA,
        ],
    )(y, src_r, dst_r, zeros)


def _agg2_weights(srow_v, drow_v, ng_v, dbuf_v, wbuf_v, lanes):
    @pl.loop(0, CH // L)
    def _(g):
        @pl.loop(0, L)
        def _(j):
            e = g * L + j
            d2 = jnp.zeros((L,), jnp.float32)
            for k in range(D // L):
                ak = srow_v[e, pl.ds(k * L, L)]
                bk = drow_v[e, pl.ds(k * L, L)]
                df = ak - bk
                d2 = d2 + df * df
            plsc.store_scatter(dbuf_v, [lanes * L + j], d2)

        tot = jnp.zeros((L,), jnp.float32)
        for l in range(L):
            tot = tot + dbuf_v[pl.ds(l * L, L)]
        wbuf_v[pl.ds(g * L, L)] = jnp.exp(ng_v[...] * tot)


def _agg2_scale(srow_v, wbuf_v):
    @pl.loop(0, CH)
    def _(e):
        we = plsc.load_gather(wbuf_v, [jnp.broadcast_to(e, (L,))])
        for k in range(D // L):
            srow_v[e, pl.ds(k * L, L)] = srow_v[e, pl.ds(k * L, L)] * we


def _agg2_body(x_hbm, src_hbm, dst_hbm, zeros_hbm, ng_hbm, aggp_hbm,
               sidx_v, didx_v, srow0_v, srow1_v, drow_v, ng_v, dbuf_v,
               wbuf_v, agg_sp, semS0, semS1, semD, semM0, semM1):
    c = lax.axis_index("c")
    s = lax.axis_index("s")
    w = c * NS + s

    pltpu.sync_copy(zeros_hbm.at[pl.ds(s * RPS, RPS)],
                    agg_sp.at[pl.ds(s * RPS, RPS)])
    plsc.subcore_barrier()

    pltpu.sync_copy(ng_hbm, ng_v)

    lanes = lax.iota(jnp.int32, L)

    @pl.loop(0, NIB)
    def _(ob):
        pltpu.sync_copy(src_hbm.at[w, ob], sidx_v)
        pltpu.sync_copy(dst_hbm.at[w, ob], didx_v)

        srows = (srow0_v, srow1_v)
        semSs = (semS0, semS1)
        semMs = (semM0, semM1)

        cpS = {0: pltpu.async_copy(x_hbm.at[sidx_v.at[0]], srows[0],
                                   semSs[0])}
        cpD = pltpu.async_copy(x_hbm.at[didx_v.at[0]], drow_v, semD)
        scat = None
        for i in range(IB):
            sr = srows[i % 2]
            cpS[i].wait()
            cpD.wait()
            _agg2_weights(sr, drow_v, ng_v, dbuf_v, wbuf_v, lanes)
            if i + 1 < IB:
                cpD = pltpu.async_copy(x_hbm.at[didx_v.at[i + 1]], drow_v,
                                       semD)
            if scat is not None:
                scat.wait()
            if i + 1 < IB:
                cpS[i + 1] = pltpu.async_copy(
                    x_hbm.at[sidx_v.at[i + 1]], srows[(i + 1) % 2],
                    semSs[(i + 1) % 2])
            _agg2_scale(sr, wbuf_v)
            scat = pltpu.async_copy(sr, agg_sp.at[didx_v.at[i]],
                                    semMs[i % 2], add=True)
        scat.wait()

    plsc.subcore_barrier()
    pltpu.sync_copy(agg_sp.at[pl.ds(s * RPS, RPS)],
                    aggp_hbm.at[pl.ds(c * NP + s * RPS, RPS)])


def _sc_agg2(x1, src_r, dst_r, zeros, ngamma):
    return pl.kernel(
        _agg2_body,
        out_type=jax.ShapeDtypeStruct((NC * NP, D), jnp.float32),
        mesh=_mesh(),
        compiler_params=pltpu.CompilerParams(needs_layout_passes=False),
        scratch_types=[
            pltpu.VMEM((IB, CH), jnp.int32),
            pltpu.VMEM((IB, CH), jnp.int32),
            pltpu.VMEM((CH, D), jnp.float32),
            pltpu.VMEM((CH, D), jnp.float32),
            pltpu.VMEM((CH, D), jnp.float32),
            pltpu.VMEM((L,), jnp.float32),
            pltpu.VMEM((L * L,), jnp.float32),
            pltpu.VMEM((CH,), jnp.float32),
            pltpu.VMEM_SHARED((NP, D), jnp.float32),
            pltpu.SemaphoreType.DMA,
            pltpu.SemaphoreType.DMA,
            pltpu.SemaphoreType.DMA,
            pltpu.SemaphoreType.DMA,
            pltpu.SemaphoreType.DMA,
        ],
    )(x1, src_r, dst_r, zeros, ngamma)


_BPW = 1024 // NW


def _sel_body(z_hbm, bidx_hbm, out_hbm, bidx_v, rows_v, sem):
    c = lax.axis_index("c")
    s = lax.axis_index("s")
    w = c * NS + s
    base = w * _BPW
    pltpu.sync_copy(bidx_hbm.at[pl.ds(base, _BPW)], bidx_v)
    pltpu.async_copy(z_hbm.at[bidx_v], rows_v, sem).wait()
    pltpu.sync_copy(rows_v, out_hbm.at[pl.ds(base, _BPW)])


def _sc_sel(z2, batch_nodes):
    return pl.kernel(
        _sel_body,
        out_type=jax.ShapeDtypeStruct((1024, D), jnp.float32),
        mesh=_mesh(),
        compiler_params=pltpu.CompilerParams(needs_layout_passes=False),
        scratch_types=[
            pltpu.VMEM((_BPW,), jnp.int32),
            pltpu.VMEM((_BPW, D), jnp.float32),
            pltpu.SemaphoreType.DMA,
        ],
    )(z2, batch_nodes)


def _prep1_body(deg_ref, x_ref, y_ref):
    d = jnp.maximum(deg_ref[0] + deg_ref[1], 1.0)
    s = lax.rsqrt(d)
    y_ref[...] = x_ref[...] * s


def _tc_prep1(degp, xp):
    return pl.pallas_call(
        _prep1_body,
        grid=(GRID,),
        in_specs=[
            pl.BlockSpec((NC, BLK, 1), lambda i: (0, i, 0)),
            pl.BlockSpec((BLK, D), lambda i: (i, 0)),
        ],
        out_specs=pl.BlockSpec((BLK, D), lambda i: (i, 0)),
        out_shape=jax.ShapeDtypeStruct((NP, D), jnp.float32),
    )(degp, xp)


def _mm_bn_body(deg_ref, agg_ref, x_ref, w_ref, b_ref, g_ref, z_ref, st_ref,
                *, self_scaled):
    i = pl.program_id(0)
    agg = agg_ref[0] + agg_ref[1]
    if self_scaled:
        d = jnp.maximum(deg_ref[0] + deg_ref[1], 1.0)
        s = lax.rsqrt(d)
        pre = g_ref[0, 0] * s * agg + x_ref[...]
    else:
        pre = agg + x_ref[...]
    z = jnp.dot(pre, w_ref[...], preferred_element_type=jnp.float32)
    z = z + b_ref[...]
    z_ref[...] = z

    rows = i * BLK + lax.broadcasted_iota(jnp.int32, (BLK, 1), 0)
    m = (rows < N).astype(jnp.float32)
    zm = z * m
    sums = jnp.sum(zm, axis=0, keepdims=True)
    sqs = jnp.sum(zm * zm, axis=0, keepdims=True)

    @pl.when(i == 0)
    def _():
        st_ref[...] = jnp.zeros_like(st_ref)

    st_ref[...] += jnp.concatenate([sums, sqs], axis=0)


def _tc_mm_bn(degp, aggp, x, W, b, g, self_scaled):
    body = functools.partial(_mm_bn_body, self_scaled=self_scaled)
    return pl.pallas_call(
        body,
        grid=(GRID,),
        in_specs=[
            pl.BlockSpec((NC, BLK, 1), lambda i: (0, i, 0)),
            pl.BlockSpec((NC, BLK, D), lambda i: (0, i, 0)),
            pl.BlockSpec((BLK, D), lambda i: (i, 0)),
            pl.BlockSpec((D, D), lambda i: (0, 0)),
            pl.BlockSpec((1, D), lambda i: (0, 0)),
            pl.BlockSpec((1, 1), lambda i: (0, 0)),
        ],
        out_specs=[
            pl.BlockSpec((BLK, D), lambda i: (i, 0)),
            pl.BlockSpec((2, D), lambda i: (0, 0)),
        ],
        out_shape=[
            jax.ShapeDtypeStruct((NP, D), jnp.float32),
            jax.ShapeDtypeStruct((2, D), jnp.float32),
        ],
    )(degp, aggp, x, W, b, g)


def _bn_relu_body(z_ref, st_ref, sc_ref, bi_ref, o_ref):
    mu = st_ref[0:1, :] / N
    var = st_ref[1:2, :] / N - mu * mu
    inv = lax.rsqrt(var + 1e-5)
    h = (z_ref[...] - mu) * inv * sc_ref[...] + bi_ref[...]
    o_ref[...] = jnp.maximum(h, 0.0)


def _tc_bn_relu(z, st, scale, bias):
    return pl.pallas_call(
        _bn_relu_body,
        grid=(GRID,),
        in_specs=[
            pl.BlockSpec((BLK, D), lambda i: (i, 0)),
            pl.BlockSpec((2, D), lambda i: (0, 0)),
            pl.BlockSpec((1, D), lambda i: (0, 0)),
            pl.BlockSpec((1, D), lambda i: (0, 0)),
        ],
        out_specs=pl.BlockSpec((BLK, D), lambda i: (i, 0)),
        out_shape=jax.ShapeDtypeStruct((NP, D), jnp.float32),
    )(z, st, scale, bias)


def _final_body(z_ref, st_ref, sc_ref, bi_ref, o_ref):
    mu = st_ref[0:1, :] / N
    var = st_ref[1:2, :] / N - mu * mu
    inv = lax.rsqrt(var + 1e-5)
    h = (z_ref[...] - mu) * inv * sc_ref[...] + bi_ref[...]
    h = jnp.maximum(h, 0.0)
    mx = jnp.max(h, axis=1, keepdims=True)
    ex = jnp.exp(h - mx)
    lse = jnp.log(jnp.sum(ex, axis=1, keepdims=True))
    o_ref[...] = h - mx - lse


def _tc_final(zsel, st, scale, bias):
    return pl.pallas_call(
        _final_body,
        grid=(2,),
        in_specs=[
            pl.BlockSpec((BLK, D), lambda i: (i, 0)),
            pl.BlockSpec((2, D), lambda i: (0, 0)),
            pl.BlockSpec((1, D), lambda i: (0, 0)),
            pl.BlockSpec((1, D), lambda i: (0, 0)),
        ],
        out_specs=pl.BlockSpec((BLK, D), lambda i: (i, 0)),
        out_shape=jax.ShapeDtypeStruct((1024, D), jnp.float32),
    )(zsel, st, scale, bias)


def kernel(features, edge_index, batch_nodes, W1, b1, gamma1, W2, b2, gamma2,
           bn1_scale, bn1_bias, bn2_scale, bn2_bias):
    f32 = jnp.float32
    xp = jnp.zeros((NP, D), f32).at[:N].set(features)
    src_r = edge_index[0].astype(jnp.int32).reshape(NW, NIB, IB, CH)
    dst_r = edge_index[1].astype(jnp.int32).reshape(NW, NIB, IB, CH)
    zeros = jnp.zeros((NP, D), f32)
    g1 = jnp.reshape(gamma1.astype(f32), (1, 1))
    ngamma2 = jnp.broadcast_to(-gamma2.astype(f32), (L,))
    sc1 = jnp.reshape(bn1_scale, (1, D))
    bi1 = jnp.reshape(bn1_bias, (1, D))
    sc2 = jnp.reshape(bn2_scale, (1, D))
    bi2 = jnp.reshape(bn2_bias, (1, D))
    b1r = jnp.reshape(b1, (1, D))
    b2r = jnp.reshape(b2, (1, D))

    degp = _sc_deg(dst_r).reshape(NC, NP, 1)
    y = _tc_prep1(degp, xp)
    aggp = _sc_agg1(y, src_r, dst_r, zeros).reshape(NC, NP, D)
    z1, st1 = _tc_mm_bn(degp, aggp, xp, W1, b1r, g1, True)
    x1 = _tc_bn_relu(z1, st1, sc1, bi1)
    aggp2 = _sc_agg2(x1, src_r, dst_r, zeros, ngamma2).reshape(NC, NP, D)
    z2, st2 = _tc_mm_bn(degp, aggp2, x1, W2, b2r, g1, False)
    zsel = _sc_sel(z2, batch_nodes.astype(jnp.int32))
    return _tc_final(zsel, st2, sc2, bi2)


# speedup vs baseline: 13.8926x; 1.0984x over previous
import functools

import jax
import jax.numpy as jnp
from jax import lax
from jax.experimental import pallas as pl
from jax.experimental.pallas import tpu as pltpu
from jax.experimental.pallas import tpu_sc as plsc

N = 10000
NP = 10240
E = 320000
D = 128
NC = 2
NS = 16
L = 16
NW = NC * NS
EPW = E // NW
CH = 80
NCHUNK = EPW // CH
RPS = NP // NS
IB = 25
NIB = NCHUNK // IB
BLK = 512
GRID = NP // BLK

@functools.cache
def _mesh():
    return plsc.VectorSubcoreMesh(core_axis_name="c", subcore_axis_name="s",
                                  num_cores=NC, num_subcores=NS)


def _deg_body(dst_hbm, degp_hbm, didx_v, ones_v, zb_v, deg_sp):
    c = lax.axis_index("c")
    s = lax.axis_index("s")
    w = c * NS + s

    @pl.loop(0, CH // L)
    def _(i):
        ones_v[pl.ds(i * L, L)] = jnp.ones((L,), jnp.float32)

    @pl.loop(0, RPS // L)
    def _(i):
        zb_v[pl.ds(i * L, L)] = jnp.zeros((L,), jnp.float32)

    pltpu.sync_copy(zb_v, deg_sp.at[pl.ds(s * RPS, RPS)])
    plsc.subcore_barrier()

    @pl.loop(0, NIB)
    def _(ob):
        pltpu.sync_copy(dst_hbm.at[w, ob], didx_v)

        @pl.loop(0, IB)
        def _(i):
            pltpu.sync_copy(ones_v, deg_sp.at[didx_v.at[i]], add=True)

    plsc.subcore_barrier()
    pltpu.sync_copy(deg_sp.at[pl.ds(s * RPS, RPS)],
                    degp_hbm.at[pl.ds(c * NP + s * RPS, RPS)])


def _sc_deg(dst_r):
    return pl.kernel(
        _deg_body,
        out_type=jax.ShapeDtypeStruct((NC * NP,), jnp.float32),
        mesh=_mesh(),
        compiler_params=pltpu.CompilerParams(needs_layout_passes=False),
        scratch_types=[
            pltpu.VMEM((IB, CH), jnp.int32),
            pltpu.VMEM((CH,), jnp.float32),
            pltpu.VMEM((RPS,), jnp.float32),
            pltpu.VMEM_SHARED((NP,), jnp.float32),
        ],
    )(dst_r)


def _agg1_body(y_hbm, src_hbm, dst_hbm, zeros_hbm, aggp_hbm,
               sidx_v, didx_v, rows0_v, rows1_v, agg_sp, sem0, sem1, semA):
    c = lax.axis_index("c")
    s = lax.axis_index("s")
    w = c * NS + s

    pltpu.sync_copy(zeros_hbm.at[pl.ds(s * RPS, RPS)],
                    agg_sp.at[pl.ds(s * RPS, RPS)])
    plsc.subcore_barrier()

    rows = (rows0_v, rows1_v)
    sems = (sem0, sem1)

    @pl.loop(0, NIB)
    def _(ob):
        pltpu.sync_copy(src_hbm.at[w, ob], sidx_v)
        pltpu.sync_copy(dst_hbm.at[w, ob], didx_v)

        cps = {0: pltpu.async_copy(y_hbm.at[sidx_v.at[0]], rows[0], sems[0])}
        for i in range(IB):
            if i + 1 < IB:
                cps[i + 1] = pltpu.async_copy(
                    y_hbm.at[sidx_v.at[i + 1]], rows[(i + 1) % 2],
                    sems[(i + 1) % 2])
            cps[i].wait()
            pltpu.sync_copy(rows[i % 2], agg_sp.at[didx_v.at[i]], add=True)

    plsc.subcore_barrier()
    pltpu.sync_copy(agg_sp.at[pl.ds(s * RPS, RPS)],
                    aggp_hbm.at[pl.ds(c * NP + s * RPS, RPS)])


def _sc_agg1(y, src_r, dst_r, zeros):
    return pl.kernel(
        _agg1_body,
        out_type=jax.ShapeDtypeStruct((NC * NP, D), jnp.float32),
        mesh=_mesh(),
        compiler_params=pltpu.CompilerParams(needs_layout_passes=False),
        scratch_types=[
            pltpu.VMEM((IB, CH), jnp.int32),
            pltpu.VMEM((IB, CH), jnp.int32),
            pltpu.VMEM((CH, D), jnp.float32),
            pltpu.VMEM((CH, D), jnp.float32),
            pltpu.VMEM_SHARED((NP, D), jnp.float32),
            pltpu.SemaphoreType.DMA,
            pltpu.SemaphoreType.DMA,
            pltpu.SemaphoreType.DMA,
        ],
    )(y, src_r, dst_r, zeros)


def _agg2_weights(srow_v, drow_v, ng_v, dbuf_v, wbuf_v, lanes):
    @pl.loop(0, CH // L)
    def _(g):
        @plsc.parallel_loop(0, L, unroll=2)
        def _(j):
            e = g * L + j
            d2 = jnp.zeros((L,), jnp.float32)
            for k in range(D // L):
                ak = srow_v[e, pl.ds(k * L, L)]
                bk = drow_v[e, pl.ds(k * L, L)]
                df = ak - bk
                d2 = d2 + df * df
            plsc.store_scatter(dbuf_v, [lanes * L + j], d2)

        tot = jnp.zeros((L,), jnp.float32)
        for l in range(L):
            tot = tot + dbuf_v[pl.ds(l * L, L)]
        wbuf_v[pl.ds(g * L, L)] = jnp.exp(ng_v[...] * tot)


def _agg2_scale(srow_v, wbuf_v):
    @plsc.parallel_loop(0, CH, unroll=2)
    def _(e):
        we = plsc.load_gather(wbuf_v, [jnp.broadcast_to(e, (L,))])
        for k in range(D // L):
            srow_v[e, pl.ds(k * L, L)] = srow_v[e, pl.ds(k * L, L)] * we


def _agg2_body(x_hbm, src_hbm, dst_hbm, zeros_hbm, ng_hbm, aggp_hbm,
               sidx_v, didx_v, srow0_v, srow1_v, drow_v, ng_v, dbuf_v,
               wbuf_v, agg_sp, semS0, semS1, semD, semM0, semM1):
    c = lax.axis_index("c")
    s = lax.axis_index("s")
    w = c * NS + s

    pltpu.sync_copy(zeros_hbm.at[pl.ds(s * RPS, RPS)],
                    agg_sp.at[pl.ds(s * RPS, RPS)])
    plsc.subcore_barrier()

    pltpu.sync_copy(ng_hbm, ng_v)

    lanes = lax.iota(jnp.int32, L)

    @pl.loop(0, NIB)
    def _(ob):
        pltpu.sync_copy(src_hbm.at[w, ob], sidx_v)
        pltpu.sync_copy(dst_hbm.at[w, ob], didx_v)

        srows = (srow0_v, srow1_v)
        semSs = (semS0, semS1)
        semMs = (semM0, semM1)

        cpS = {0: pltpu.async_copy(x_hbm.at[sidx_v.at[0]], srows[0],
                                   semSs[0])}
        cpD = pltpu.async_copy(x_hbm.at[didx_v.at[0]], drow_v, semD)
        scat = None
        for i in range(IB):
            sr = srows[i % 2]
            cpS[i].wait()
            cpD.wait()
            _agg2_weights(sr, drow_v, ng_v, dbuf_v, wbuf_v, lanes)
            if i + 1 < IB:
                cpD = pltpu.async_copy(x_hbm.at[didx_v.at[i + 1]], drow_v,
                                       semD)
            if scat is not None:
                scat.wait()
            if i + 1 < IB:
                cpS[i + 1] = pltpu.async_copy(
                    x_hbm.at[sidx_v.at[i + 1]], srows[(i + 1) % 2],
                    semSs[(i + 1) % 2])
            _agg2_scale(sr, wbuf_v)
            scat = pltpu.async_copy(sr, agg_sp.at[didx_v.at[i]],
                                    semMs[i % 2], add=True)
        scat.wait()

    plsc.subcore_barrier()
    pltpu.sync_copy(agg_sp.at[pl.ds(s * RPS, RPS)],
                    aggp_hbm.at[pl.ds(c * NP + s * RPS, RPS)])


def _sc_agg2(x1, src_r, dst_r, zeros, ngamma):
    return pl.kernel(
        _agg2_body,
        out_type=jax.ShapeDtypeStruct((NC * NP, D), jnp.float32),
        mesh=_mesh(),
        compiler_params=pltpu.CompilerParams(needs_layout_passes=False),
        scratch_types=[
            pltpu.VMEM((IB, CH), jnp.int32),
            pltpu.VMEM((IB, CH), jnp.int32),
            pltpu.VMEM((CH, D), jnp.float32),
            pltpu.VMEM((CH, D), jnp.float32),
            pltpu.VMEM((CH, D), jnp.float32),
            pltpu.VMEM((L,), jnp.float32),
            pltpu.VMEM((L * L,), jnp.float32),
            pltpu.VMEM((CH,), jnp.float32),
            pltpu.VMEM_SHARED((NP, D), jnp.float32),
            pltpu.SemaphoreType.DMA,
            pltpu.SemaphoreType.DMA,
            pltpu.SemaphoreType.DMA,
            pltpu.SemaphoreType.DMA,
            pltpu.SemaphoreType.DMA,
        ],
    )(x1, src_r, dst_r, zeros, ngamma)


_BPW = 1024 // NW


def _sel_body(z_hbm, bidx_hbm, out_hbm, bidx_v, rows_v, sem):
    c = lax.axis_index("c")
    s = lax.axis_index("s")
    w = c * NS + s
    base = w * _BPW
    pltpu.sync_copy(bidx_hbm.at[pl.ds(base, _BPW)], bidx_v)
    pltpu.async_copy(z_hbm.at[bidx_v], rows_v, sem).wait()
    pltpu.sync_copy(rows_v, out_hbm.at[pl.ds(base, _BPW)])


def _sc_sel(z2, batch_nodes):
    return pl.kernel(
        _sel_body,
        out_type=jax.ShapeDtypeStruct((1024, D), jnp.float32),
        mesh=_mesh(),
        compiler_params=pltpu.CompilerParams(needs_layout_passes=False),
        scratch_types=[
            pltpu.VMEM((_BPW,), jnp.int32),
            pltpu.VMEM((_BPW, D), jnp.float32),
            pltpu.SemaphoreType.DMA,
        ],
    )(z2, batch_nodes)


def _prep1_body(deg_ref, x_ref, y_ref):
    d = jnp.maximum(deg_ref[0] + deg_ref[1], 1.0)
    s = lax.rsqrt(d)
    y_ref[...] = x_ref[...] * s


def _tc_prep1(degp, xp):
    return pl.pallas_call(
        _prep1_body,
        grid=(GRID,),
        in_specs=[
            pl.BlockSpec((NC, BLK, 1), lambda i: (0, i, 0)),
            pl.BlockSpec((BLK, D), lambda i: (i, 0)),
        ],
        out_specs=pl.BlockSpec((BLK, D), lambda i: (i, 0)),
        out_shape=jax.ShapeDtypeStruct((NP, D), jnp.float32),
    )(degp, xp)


def _mm_bn_body(deg_ref, agg_ref, x_ref, w_ref, b_ref, g_ref, z_ref, st_ref,
                *, self_scaled):
    i = pl.program_id(0)
    agg = agg_ref[0] + agg_ref[1]
    if self_scaled:
        d = jnp.maximum(deg_ref[0] + deg_ref[1], 1.0)
        s = lax.rsqrt(d)
        pre = g_ref[0, 0] * s * agg + x_ref[...]
    else:
        pre = agg + x_ref[...]
    z = jnp.dot(pre, w_ref[...], preferred_element_type=jnp.float32)
    z = z + b_ref[...]
    z_ref[...] = z

    rows = i * BLK + lax.broadcasted_iota(jnp.int32, (BLK, 1), 0)
    m = (rows < N).astype(jnp.float32)
    zm = z * m
    sums = jnp.sum(zm, axis=0, keepdims=True)
    sqs = jnp.sum(zm * zm, axis=0, keepdims=True)

    @pl.when(i == 0)
    def _():
        st_ref[...] = jnp.zeros_like(st_ref)

    st_ref[...] += jnp.concatenate([sums, sqs], axis=0)


def _tc_mm_bn(degp, aggp, x, W, b, g, self_scaled):
    body = functools.partial(_mm_bn_body, self_scaled=self_scaled)
    return pl.pallas_call(
        body,
        grid=(GRID,),
        in_specs=[
            pl.BlockSpec((NC, BLK, 1), lambda i: (0, i, 0)),
            pl.BlockSpec((NC, BLK, D), lambda i: (0, i, 0)),
            pl.BlockSpec((BLK, D), lambda i: (i, 0)),
            pl.BlockSpec((D, D), lambda i: (0, 0)),
            pl.BlockSpec((1, D), lambda i: (0, 0)),
            pl.BlockSpec((1, 1), lambda i: (0, 0)),
        ],
        out_specs=[
            pl.BlockSpec((BLK, D), lambda i: (i, 0)),
            pl.BlockSpec((2, D), lambda i: (0, 0)),
        ],
        out_shape=[
            jax.ShapeDtypeStruct((NP, D), jnp.float32),
            jax.ShapeDtypeStruct((2, D), jnp.float32),
        ],
    )(degp, aggp, x, W, b, g)


def _bn_relu_body(z_ref, st_ref, sc_ref, bi_ref, o_ref):
    mu = st_ref[0:1, :] / N
    var = st_ref[1:2, :] / N - mu * mu
    inv = lax.rsqrt(var + 1e-5)
    h = (z_ref[...] - mu) * inv * sc_ref[...] + bi_ref[...]
    o_ref[...] = jnp.maximum(h, 0.0)


def _tc_bn_relu(z, st, scale, bias):
    return pl.pallas_call(
        _bn_relu_body,
        grid=(GRID,),
        in_specs=[
            pl.BlockSpec((BLK, D), lambda i: (i, 0)),
            pl.BlockSpec((2, D), lambda i: (0, 0)),
            pl.BlockSpec((1, D), lambda i: (0, 0)),
            pl.BlockSpec((1, D), lambda i: (0, 0)),
        ],
        out_specs=pl.BlockSpec((BLK, D), lambda i: (i, 0)),
        out_shape=jax.ShapeDtypeStruct((NP, D), jnp.float32),
    )(z, st, scale, bias)


def _final_body(z_ref, st_ref, sc_ref, bi_ref, o_ref):
    mu = st_ref[0:1, :] / N
    var = st_ref[1:2, :] / N - mu * mu
    inv = lax.rsqrt(var + 1e-5)
    h = (z_ref[...] - mu) * inv * sc_ref[...] + bi_ref[...]
    h = jnp.maximum(h, 0.0)
    mx = jnp.max(h, axis=1, keepdims=True)
    ex = jnp.exp(h - mx)
    lse = jnp.log(jnp.sum(ex, axis=1, keepdims=True))
    o_ref[...] = h - mx - lse


def _tc_final(zsel, st, scale, bias):
    return pl.pallas_call(
        _final_body,
        grid=(2,),
        in_specs=[
            pl.BlockSpec((BLK, D), lambda i: (i, 0)),
            pl.BlockSpec((2, D), lambda i: (0, 0)),
            pl.BlockSpec((1, D), lambda i: (0, 0)),
            pl.BlockSpec((1, D), lambda i: (0, 0)),
        ],
        out_specs=pl.BlockSpec((BLK, D), lambda i: (i, 0)),
        out_shape=jax.ShapeDtypeStruct((1024, D), jnp.float32),
    )(zsel, st, scale, bias)


def kernel(features, edge_index, batch_nodes, W1, b1, gamma1, W2, b2, gamma2,
           bn1_scale, bn1_bias, bn2_scale, bn2_bias):
    f32 = jnp.float32
    xp = jnp.zeros((NP, D), f32).at[:N].set(features)
    src_r = edge_index[0].astype(jnp.int32).reshape(NW, NIB, IB, CH)
    dst_r = edge_index[1].astype(jnp.int32).reshape(NW, NIB, IB, CH)
    zeros = jnp.zeros((NP, D), f32)
    g1 = jnp.reshape(gamma1.astype(f32), (1, 1))
    ngamma2 = jnp.broadcast_to(-gamma2.astype(f32), (L,))
    sc1 = jnp.reshape(bn1_scale, (1, D))
    bi1 = jnp.reshape(bn1_bias, (1, D))
    sc2 = jnp.reshape(bn2_scale, (1, D))
    bi2 = jnp.reshape(bn2_bias, (1, D))
    b1r = jnp.reshape(b1, (1, D))
    b2r = jnp.reshape(b2, (1, D))

    degp = _sc_deg(dst_r).reshape(NC, NP, 1)
    y = _tc_prep1(degp, xp)
    aggp = _sc_agg1(y, src_r, dst_r, zeros).reshape(NC, NP, D)
    z1, st1 = _tc_mm_bn(degp, aggp, xp, W1, b1r, g1, True)
    x1 = _tc_bn_relu(z1, st1, sc1, bi1)
    aggp2 = _sc_agg2(x1, src_r, dst_r, zeros, ngamma2).reshape(NC, NP, D)
    z2, st2 = _tc_mm_bn(degp, aggp2, x1, W2, b2r, g1, False)
    zsel = _sc_sel(z2, batch_nodes.astype(jnp.int32))
    return _tc_final(zsel, st2, sc2, bi2)
